# baseline (speedup 1.0000x reference)
import functools

import jax
import jax.numpy as jnp
from jax import lax
from jax.experimental import pallas as pl
from jax.experimental.pallas import tpu as pltpu
from jax.experimental.pallas import tpu_sc as plsc

D_MODEL = 1024
D_FF = 2048
E = 8
TOPK = 2
T = 2048
CAP = 640
LE = 128
NW = 32
TPW = T // NW
TRASH = E * CAP
DISP_ROWS = 5760
WSL = 128
CH = 256


def _router_body(x_ref, wr_ref, slots_ref, aux_ref, w1r_ref, w2r_ref,
                 a_scr, s_scr):
    x = x_ref[...]
    wr = wr_ref[...]
    logits = jnp.dot(x, wr, preferred_element_type=jnp.float32)
    lane = lax.broadcasted_iota(jnp.int32, (T, LE), 1)
    valid = lane < E
    neg = jnp.float32(-1e30)
    logits = jnp.where(valid, logits, neg)
    m1 = jnp.max(logits, axis=1, keepdims=True)
    i1 = jnp.min(jnp.where(logits == m1, lane, LE), axis=1, keepdims=True)
    mask1 = lane == i1
    l2 = jnp.where(mask1, neg, logits)
    m2 = jnp.max(l2, axis=1, keepdims=True)
    i2 = jnp.min(jnp.where(l2 == m2, lane, LE), axis=1, keepdims=True)
    mask2 = lane == i2
    p = jnp.where(valid, jnp.exp(logits - m1), 0.0)
    psum = jnp.sum(p, axis=1, keepdims=True)
    probs = p / psum
    p1 = jnp.sum(jnp.where(mask1, probs, 0.0), axis=1, keepdims=True)
    p2 = jnp.sum(jnp.where(mask2, probs, 0.0), axis=1, keepdims=True)
    wsum = p1 + p2
    w1 = p1 / wsum
    w2 = p2 / wsum
    a = mask1.astype(jnp.float32) + mask2.astype(jnp.float32)
    a_scr[...] = a
    ii = lax.broadcasted_iota(jnp.int32, (CH, CH), 0)
    jj = lax.broadcasted_iota(jnp.int32, (CH, CH), 1)
    tril = (jj < ii).astype(jnp.float32)

    def chunk(c, run):
        a_c = a_scr[pl.ds(c * CH, CH), :]
        s_scr[pl.ds(c * CH, CH), :] = (
            jnp.dot(tril, a_c, preferred_element_type=jnp.float32) + run)
        return run + jnp.sum(a_c, axis=0, keepdims=True)

    lax.fori_loop(0, T // CH, chunk, jnp.zeros((1, LE), jnp.float32))
    s = s_scr[...]
    pos1 = jnp.sum(jnp.where(mask1, s, 0.0), axis=1, keepdims=True)
    pos2 = jnp.sum(jnp.where(mask2, s, 0.0), axis=1, keepdims=True)
    keep1 = pos1 < CAP
    keep2 = pos2 < CAP
    pos1c = jnp.minimum(pos1, CAP - 1).astype(jnp.int32)
    pos2c = jnp.minimum(pos2, CAP - 1).astype(jnp.int32)
    slotd1 = jnp.where(keep1, i1 * CAP + pos1c, TRASH)
    slotd2 = jnp.where(keep2, i2 * CAP + pos2c, TRASH)
    slotc1 = jnp.where(keep1, i1 * CAP + pos1c, TRASH)
    slotc2 = jnp.where(keep2, i2 * CAP + pos2c, TRASH)
    w1k = jnp.where(keep1, w1, 0.0)
    w2k = jnp.where(keep2, w2, 0.0)
    ce = jnp.sum(a, axis=0, keepdims=True) / T
    me = jnp.sum(probs, axis=0, keepdims=True) / T
    aux_ref[...] = jnp.reshape((E / TOPK) * jnp.sum(me * ce), (1, 1))
    lane8 = lax.broadcasted_iota(jnp.int32, (T, 8), 1)
    packed = jnp.where(lane8 == 0, slotd1,
             jnp.where(lane8 == 1, slotd2,
             jnp.where(lane8 == 2, slotc1,
             jnp.where(lane8 == 3, slotc2,
                       0))))
    slots_ref[...] = jnp.transpose(packed)
    w1r_ref[...] = jnp.broadcast_to(w1k, (T, WSL))
    w2r_ref[...] = jnp.broadcast_to(w2k, (T, WSL))


_router = pl.pallas_call(
    _router_body,
    out_shape=(jax.ShapeDtypeStruct((8, T), jnp.int32),
               jax.ShapeDtypeStruct((1, 1), jnp.float32),
               jax.ShapeDtypeStruct((T, WSL), jnp.float32),
               jax.ShapeDtypeStruct((T, WSL), jnp.float32)),
    scratch_shapes=[pltpu.VMEM((T, LE), jnp.float32),
                    pltpu.VMEM((T, LE), jnp.float32)],
)


def _dispatch_body(x_hbm, slots_hbm, w1r_hbm, w2r_hbm, disp_hbm, ws_hbm,
                   idx1_v, idx2_v, xbuf, w1buf, w2buf, sem1, sem2, sem3):
    wid = lax.axis_index("s") * 2 + lax.axis_index("c")
    base = wid * TPW
    pltpu.sync_copy(slots_hbm.at[0, pl.ds(base, TPW)], idx1_v)
    pltpu.sync_copy(slots_hbm.at[1, pl.ds(base, TPW)], idx2_v)
    cp = pltpu.async_copy(x_hbm.at[pl.ds(base, TPW)], xbuf, sem3)
    cw1 = pltpu.async_copy(w1r_hbm.at[pl.ds(base, TPW)], w1buf, sem1)
    cw2 = pltpu.async_copy(w2r_hbm.at[pl.ds(base, TPW)], w2buf, sem2)
    for i in range(TPW // 16):
        sl = pl.ds(i * 16, 16)
        v1 = idx1_v[sl]
        idx1_v[sl] = jnp.where(v1 >= TRASH, TRASH + wid, v1)
        v2 = idx2_v[sl]
        idx2_v[sl] = jnp.where(v2 >= TRASH, TRASH + wid, v2)
    cp.wait()
    cw1.wait()
    cw2.wait()
    c1 = pltpu.async_copy(xbuf, disp_hbm.at[idx1_v], sem1)
    c2 = pltpu.async_copy(xbuf, disp_hbm.at[idx2_v], sem2)
    c3 = pltpu.async_copy(w1buf, ws_hbm.at[idx1_v], sem3)
    c1.wait()
    c2.wait()
    c3.wait()
    c4 = pltpu.async_copy(w2buf, ws_hbm.at[idx2_v], sem3)
    c4.wait()


def _ffn_body(disp_ref, w1_ref, w2_ref, ws_ref, y_ref):
    e = pl.program_id(0)

    @pl.when(e == 0)
    def _zero():
        y_ref[...] = jnp.zeros((CAP, D_MODEL), jnp.float32)

    @pl.when(e > 0)
    def _compute():
        def m_body(m, carry):
            xm = disp_ref[pl.ds(m * 128, 128), :]
            h = jnp.maximum(
                jnp.dot(xm, w1_ref[0], preferred_element_type=jnp.float32),
                0.0)
            y = jnp.dot(h, w2_ref[0], preferred_element_type=jnp.float32)
            wrow = ws_ref[pl.ds(m * 128, 128), 0:1]
            y_ref[pl.ds(m * 128, 128), :] = y * wrow
            return carry

        lax.fori_loop(0, CAP // 128, m_body, 0)


def _in_map(e, *_):
    return (jnp.maximum(e - 1, 0), 0)


def _in_map3(e, *_):
    return (jnp.maximum(e - 1, 0), 0, 0)


def _out_map(e, *_):
    return (jnp.where(e == 0, E, e - 1), 0)


_ffn = pl.pallas_call(
    _ffn_body,
    grid=(E + 1,),
    in_specs=[pl.BlockSpec((CAP, D_MODEL), _in_map),
              pl.BlockSpec((1, D_MODEL, D_FF), _in_map3),
              pl.BlockSpec((1, D_FF, D_MODEL), _in_map3),
              pl.BlockSpec((CAP, WSL), _in_map)],
    out_specs=pl.BlockSpec((CAP, D_MODEL), _out_map),
    out_shape=jax.ShapeDtypeStruct((DISP_ROWS, D_MODEL), jnp.float32),
)


CCH = 16


def _combine_body(y_hbm, slots_hbm, out_hbm, idx1_v, idx2_v,
                  bufs1, bufs2, sems):
    wid = lax.axis_index("s") * 2 + lax.axis_index("c")
    base = wid * TPW
    pltpu.sync_copy(slots_hbm.at[2, pl.ds(base, TPW)], idx1_v)
    pltpu.sync_copy(slots_hbm.at[3, pl.ds(base, TPW)], idx2_v)
    nch = TPW // CCH

    def issue(c, buf):
        off = c * CCH
        a1 = pltpu.async_copy(y_hbm.at[idx1_v.at[pl.ds(off, CCH)]],
                              bufs1.at[buf], sems.at[2 * buf])
        a2 = pltpu.async_copy(y_hbm.at[idx2_v.at[pl.ds(off, CCH)]],
                              bufs2.at[buf], sems.at[2 * buf + 1])
        return a1, a2

    pend = issue(0, 0)
    for c in range(nch):
        buf = c % 2
        pend[0].wait()
        pend[1].wait()
        if c + 1 < nch:
            pend = issue(c + 1, (c + 1) % 2)

        def tok(t, carry):
            for u in range(D_MODEL // 16):
                sl = pl.ds(u * 16, 16)
                bufs1[buf, t, sl] = bufs1[buf, t, sl] + bufs2[buf, t, sl]
            return carry

        lax.fori_loop(0, CCH, tok, 0)
        pltpu.sync_copy(bufs1.at[buf],
                        out_hbm.at[pl.ds(base + c * CCH, CCH)])


@functools.cache
def _sc_kernels():
    mesh = plsc.VectorSubcoreMesh(core_axis_name="c", subcore_axis_name="s")
    dispatch = pl.kernel(
        _dispatch_body, mesh=mesh,
        out_type=(jax.ShapeDtypeStruct((DISP_ROWS, D_MODEL), jnp.float32),
                  jax.ShapeDtypeStruct((DISP_ROWS, WSL), jnp.float32)),
        scratch_types=[pltpu.VMEM((TPW,), jnp.int32),
                       pltpu.VMEM((TPW,), jnp.int32),
                       pltpu.VMEM((TPW, D_MODEL), jnp.float32),
                       pltpu.VMEM((TPW, WSL), jnp.float32),
                       pltpu.VMEM((TPW, WSL), jnp.float32),
                       pltpu.SemaphoreType.DMA,
                       pltpu.SemaphoreType.DMA,
                       pltpu.SemaphoreType.DMA],
    )
    combine = pl.kernel(
        _combine_body, mesh=mesh,
        out_type=jax.ShapeDtypeStruct((T, D_MODEL), jnp.float32),
        scratch_types=[pltpu.VMEM((TPW,), jnp.int32),
                       pltpu.VMEM((TPW,), jnp.int32),
                       pltpu.VMEM((2, CCH, D_MODEL), jnp.float32),
                       pltpu.VMEM((2, CCH, D_MODEL), jnp.float32),
                       pltpu.SemaphoreType.DMA((4,))],
    )
    return dispatch, combine


def kernel(inputs, Wr, W1, W2):
    wr_pad = jnp.pad(Wr, ((0, 0), (0, LE - E)))
    slots, aux, w1r, w2r = _router(inputs, wr_pad)
    dispatch, combine = _sc_kernels()
    disp, ws = dispatch(inputs, slots, w1r, w2r)
    y = _ffn(disp, W1, W2, ws)
    out = combine(y, slots)
    return out, aux.reshape(())


# speedup vs baseline: 1.0415x; 1.0415x over previous
import functools

import jax
import jax.numpy as jnp
from jax import lax
from jax.experimental import pallas as pl
from jax.experimental.pallas import tpu as pltpu
from jax.experimental.pallas import tpu_sc as plsc

D_MODEL = 1024
D_FF = 2048
E = 8
TOPK = 2
T = 2048
CAP = 640
LE = 128
NW = 32
TPW = T // NW
TRASH = E * CAP
DISP_ROWS = 5760
WSL = 128
CH = 256


def _router_body(x_ref, wr_ref, slots_ref, aux_ref, w1r_ref, w2r_ref,
                 a_scr, s_scr):
    x = x_ref[...]
    wr = wr_ref[...]
    logits = jnp.dot(x, wr, preferred_element_type=jnp.float32)
    lane = lax.broadcasted_iota(jnp.int32, (T, LE), 1)
    valid = lane < E
    neg = jnp.float32(-1e30)
    logits = jnp.where(valid, logits, neg)
    m1 = jnp.max(logits, axis=1, keepdims=True)
    i1 = jnp.min(jnp.where(logits == m1, lane, LE), axis=1, keepdims=True)
    mask1 = lane == i1
    l2 = jnp.where(mask1, neg, logits)
    m2 = jnp.max(l2, axis=1, keepdims=True)
    i2 = jnp.min(jnp.where(l2 == m2, lane, LE), axis=1, keepdims=True)
    mask2 = lane == i2
    p = jnp.where(valid, jnp.exp(logits - m1), 0.0)
    psum = jnp.sum(p, axis=1, keepdims=True)
    probs = p / psum
    p1 = jnp.sum(jnp.where(mask1, probs, 0.0), axis=1, keepdims=True)
    p2 = jnp.sum(jnp.where(mask2, probs, 0.0), axis=1, keepdims=True)
    wsum = p1 + p2
    w1 = p1 / wsum
    w2 = p2 / wsum
    a = mask1.astype(jnp.float32) + mask2.astype(jnp.float32)
    a_scr[...] = a
    ii = lax.broadcasted_iota(jnp.int32, (CH, CH), 0)
    jj = lax.broadcasted_iota(jnp.int32, (CH, CH), 1)
    tril = (jj < ii).astype(jnp.float32)

    def chunk(c, run):
        a_c = a_scr[pl.ds(c * CH, CH), :]
        s_scr[pl.ds(c * CH, CH), :] = (
            jnp.dot(tril, a_c, preferred_element_type=jnp.float32) + run)
        return run + jnp.sum(a_c, axis=0, keepdims=True)

    lax.fori_loop(0, T // CH, chunk, jnp.zeros((1, LE), jnp.float32))
    s = s_scr[...]
    pos1 = jnp.sum(jnp.where(mask1, s, 0.0), axis=1, keepdims=True)
    pos2 = jnp.sum(jnp.where(mask2, s, 0.0), axis=1, keepdims=True)
    keep1 = pos1 < CAP
    keep2 = pos2 < CAP
    pos1c = jnp.minimum(pos1, CAP - 1).astype(jnp.int32)
    pos2c = jnp.minimum(pos2, CAP - 1).astype(jnp.int32)
    slotd1 = jnp.where(keep1, i1 * CAP + pos1c, TRASH)
    slotd2 = jnp.where(keep2, i2 * CAP + pos2c, TRASH)
    slotc1 = jnp.where(keep1, i1 * CAP + pos1c, TRASH)
    slotc2 = jnp.where(keep2, i2 * CAP + pos2c, TRASH)
    w1k = jnp.where(keep1, w1, 0.0)
    w2k = jnp.where(keep2, w2, 0.0)
    ce = jnp.sum(a, axis=0, keepdims=True) / T
    me = jnp.sum(probs, axis=0, keepdims=True) / T
    aux_ref[...] = jnp.reshape((E / TOPK) * jnp.sum(me * ce), (1, 1))
    lane8 = lax.broadcasted_iota(jnp.int32, (T, 8), 1)
    packed = jnp.where(lane8 == 0, slotd1,
             jnp.where(lane8 == 1, slotd2,
             jnp.where(lane8 == 2, slotc1,
             jnp.where(lane8 == 3, slotc2,
                       0))))
    slots_ref[...] = jnp.transpose(packed)
    w1r_ref[...] = jnp.broadcast_to(w1k, (T, WSL))
    w2r_ref[...] = jnp.broadcast_to(w2k, (T, WSL))


_router = pl.pallas_call(
    _router_body,
    out_shape=(jax.ShapeDtypeStruct((8, T), jnp.int32),
               jax.ShapeDtypeStruct((1, 1), jnp.float32),
               jax.ShapeDtypeStruct((T, WSL), jnp.float32),
               jax.ShapeDtypeStruct((T, WSL), jnp.float32)),
    scratch_shapes=[pltpu.VMEM((T, LE), jnp.float32),
                    pltpu.VMEM((T, LE), jnp.float32)],
)


def _dispatch_body(x_hbm, slots_hbm, w1r_hbm, w2r_hbm, disp_hbm, ws_hbm,
                   idx1_v, idx2_v, xbuf, w1buf, w2buf, sem1, sem2, sem3):
    wid = lax.axis_index("s") * 2 + lax.axis_index("c")
    base = wid * TPW
    pltpu.sync_copy(slots_hbm.at[0, pl.ds(base, TPW)], idx1_v)
    pltpu.sync_copy(slots_hbm.at[1, pl.ds(base, TPW)], idx2_v)
    cp = pltpu.async_copy(x_hbm.at[pl.ds(base, TPW)], xbuf, sem3)
    cw1 = pltpu.async_copy(w1r_hbm.at[pl.ds(base, TPW)], w1buf, sem1)
    cw2 = pltpu.async_copy(w2r_hbm.at[pl.ds(base, TPW)], w2buf, sem2)
    for i in range(TPW // 16):
        sl = pl.ds(i * 16, 16)
        v1 = idx1_v[sl]
        idx1_v[sl] = jnp.where(v1 >= TRASH, TRASH + wid, v1)
        v2 = idx2_v[sl]
        idx2_v[sl] = jnp.where(v2 >= TRASH, TRASH + wid, v2)
    cp.wait()
    cw1.wait()
    cw2.wait()
    c1 = pltpu.async_copy(xbuf, disp_hbm.at[idx1_v], sem1)
    c2 = pltpu.async_copy(xbuf, disp_hbm.at[idx2_v], sem2)
    c3 = pltpu.async_copy(w1buf, ws_hbm.at[idx1_v], sem3)
    c1.wait()
    c2.wait()
    c3.wait()
    c4 = pltpu.async_copy(w2buf, ws_hbm.at[idx2_v], sem3)
    c4.wait()


def _ffn_body(disp_ref, w1_ref, w2_ref, ws_ref, y_ref):
    e = pl.program_id(0)

    @pl.when(e == 0)
    def _zero():
        y_ref[...] = jnp.zeros((CAP, D_MODEL), jnp.float32)

    @pl.when(e > 0)
    def _compute():
        def m_body(m, carry):
            xm = disp_ref[pl.ds(m * 128, 128), :]
            h = jnp.maximum(
                jnp.dot(xm, w1_ref[0], preferred_element_type=jnp.float32),
                0.0)
            y = jnp.dot(h, w2_ref[0], preferred_element_type=jnp.float32)
            wrow = ws_ref[pl.ds(m * 128, 128), 0:1]
            y_ref[pl.ds(m * 128, 128), :] = y * wrow
            return carry

        lax.fori_loop(0, CAP // 128, m_body, 0)


def _in_map(e, *_):
    return (jnp.maximum(e - 1, 0), 0)


def _in_map3(e, *_):
    return (jnp.maximum(e - 1, 0), 0, 0)


def _out_map(e, *_):
    return (jnp.where(e == 0, E, e - 1), 0)


_ffn = pl.pallas_call(
    _ffn_body,
    grid=(E + 1,),
    in_specs=[pl.BlockSpec((CAP, D_MODEL), _in_map),
              pl.BlockSpec((1, D_MODEL, D_FF), _in_map3),
              pl.BlockSpec((1, D_FF, D_MODEL), _in_map3),
              pl.BlockSpec((CAP, WSL), _in_map)],
    out_specs=pl.BlockSpec((CAP, D_MODEL), _out_map),
    out_shape=jax.ShapeDtypeStruct((DISP_ROWS, D_MODEL), jnp.float32),
)


CCH = 16


def _combine_body(y_hbm, slots_hbm, out_hbm, idx1_v, idx2_v,
                  bufs1, bufs2, sems):
    wid = lax.axis_index("s") * 2 + lax.axis_index("c")
    base = wid * TPW
    pltpu.sync_copy(slots_hbm.at[2, pl.ds(base, TPW)], idx1_v)
    pltpu.sync_copy(slots_hbm.at[3, pl.ds(base, TPW)], idx2_v)
    nch = TPW // CCH

    def issue(c, buf):
        off = c * CCH
        a1 = pltpu.async_copy(y_hbm.at[idx1_v.at[pl.ds(off, CCH)]],
                              bufs1.at[buf], sems.at[2 * buf])
        a2 = pltpu.async_copy(y_hbm.at[idx2_v.at[pl.ds(off, CCH)]],
                              bufs2.at[buf], sems.at[2 * buf + 1])
        return a1, a2

    pend = issue(0, 0)
    for c in range(nch):
        buf = c % 2
        pend[0].wait()
        pend[1].wait()
        if c + 1 < nch:
            pend = issue(c + 1, (c + 1) % 2)

        def tok(t, carry):
            def vv(v, carry2):
                for u in range(4):
                    sl = pl.ds((v * 4 + u) * 16, 16)
                    bufs1[buf, t, sl] = bufs1[buf, t, sl] + bufs2[buf, t, sl]
                return carry2

            lax.fori_loop(0, D_MODEL // 64, vv, 0)
            return carry

        lax.fori_loop(0, CCH, tok, 0)
        pltpu.sync_copy(bufs1.at[buf],
                        out_hbm.at[pl.ds(base + c * CCH, CCH)])


@functools.cache
def _sc_kernels():
    mesh = plsc.VectorSubcoreMesh(core_axis_name="c", subcore_axis_name="s")
    dispatch = pl.kernel(
        _dispatch_body, mesh=mesh,
        out_type=(jax.ShapeDtypeStruct((DISP_ROWS, D_MODEL), jnp.float32),
                  jax.ShapeDtypeStruct((DISP_ROWS, WSL), jnp.float32)),
        scratch_types=[pltpu.VMEM((TPW,), jnp.int32),
                       pltpu.VMEM((TPW,), jnp.int32),
                       pltpu.VMEM((TPW, D_MODEL), jnp.float32),
                       pltpu.VMEM((TPW, WSL), jnp.float32),
                       pltpu.VMEM((TPW, WSL), jnp.float32),
                       pltpu.SemaphoreType.DMA,
                       pltpu.SemaphoreType.DMA,
                       pltpu.SemaphoreType.DMA],
    )
    combine = pl.kernel(
        _combine_body, mesh=mesh,
        out_type=jax.ShapeDtypeStruct((T, D_MODEL), jnp.float32),
        scratch_types=[pltpu.VMEM((TPW,), jnp.int32),
                       pltpu.VMEM((TPW,), jnp.int32),
                       pltpu.VMEM((2, CCH, D_MODEL), jnp.float32),
                       pltpu.VMEM((2, CCH, D_MODEL), jnp.float32),
                       pltpu.SemaphoreType.DMA((4,))],
    )
    return dispatch, combine


def kernel(inputs, Wr, W1, W2):
    wr_pad = jnp.pad(Wr, ((0, 0), (0, LE - E)))
    slots, aux, w1r, w2r = _router(inputs, wr_pad)
    dispatch, combine = _sc_kernels()
    disp, ws = dispatch(inputs, slots, w1r, w2r)
    y = _ffn(disp, W1, W2, ws)
    out = combine(y, slots)
    return out, aux.reshape(())
